# baseline (speedup 1.0000x reference)
import functools

import jax
import jax.numpy as jnp
from jax.experimental import pallas as pl
from jax.experimental.pallas import tpu as pltpu

_ROWS, _COLS = 16384, 50
_HALF = _ROWS // 2
_RCHUNKS = tuple(
    (r0, 8, c0, _HALF) for r0 in range(0, 48, 8) for c0 in (0, _HALF)
) + ((48, 2, 0, _ROWS),)
_NCH = len(_RCHUNKS)


def _copy_body(x_ref, o_ref, buf, *sems):
    in_sems, out_sems = sems[:_NCH], sems[_NCH:]
    ins = [
        pltpu.make_async_copy(
            x_ref.at[pl.ds(r0, nr), pl.ds(c0, nc)],
            buf.at[pl.ds(r0, nr), pl.ds(c0, nc)],
            in_sems[k],
        )
        for k, (r0, nr, c0, nc) in enumerate(_RCHUNKS)
    ]
    outs = [
        pltpu.make_async_copy(
            buf.at[pl.ds(r0, nr), pl.ds(c0, nc)],
            o_ref.at[pl.ds(r0, nr), pl.ds(c0, nc)],
            out_sems[k],
        )
        for k, (r0, nr, c0, nc) in enumerate(_RCHUNKS)
    ]
    for c in ins:
        c.start()
    for k in range(_NCH):
        ins[k].wait()
        outs[k].start()
    for c in outs:
        c.wait()


@functools.cache
def _make_copy_kernel():
    return pl.pallas_call(
        _copy_body,
        in_specs=[pl.BlockSpec(memory_space=pl.ANY)],
        out_specs=pl.BlockSpec(memory_space=pl.ANY),
        out_shape=jax.ShapeDtypeStruct((_COLS, _ROWS), jnp.float32),
        scratch_shapes=[pltpu.VMEM((_COLS, _ROWS), jnp.float32)]
        + [pltpu.SemaphoreType.DMA] * (2 * _NCH),
    )


def kernel(candidate_titles):
    xt = pltpu.with_memory_space_constraint(
        candidate_titles.T, pltpu.MemorySpace.HBM
    )
    return _make_copy_kernel()(xt).T


# speedup vs baseline: 1.0277x; 1.0277x over previous
import functools

import jax
import jax.numpy as jnp
from jax.experimental import pallas as pl
from jax.experimental.pallas import tpu as pltpu

_ROWS, _COLS = 16384, 50
_RCHUNKS = ((0, 8), (8, 8), (16, 8), (24, 8), (32, 8), (40, 8), (48, 2))
_NCH = len(_RCHUNKS)


def _copy_body(x_ref, o_ref, buf, *sems):
    in_sems, out_sems = sems[:_NCH], sems[_NCH:]
    ins = [
        pltpu.make_async_copy(
            x_ref.at[pl.ds(r0, nr)],
            buf.at[pl.ds(r0, nr)],
            in_sems[k],
        )
        for k, (r0, nr) in enumerate(_RCHUNKS)
    ]
    outs = [
        pltpu.make_async_copy(
            buf.at[pl.ds(r0, nr)],
            o_ref.at[pl.ds(r0, nr)],
            out_sems[k],
        )
        for k, (r0, nr) in enumerate(_RCHUNKS)
    ]
    for c in ins:
        c.start()
    for k in range(_NCH):
        ins[k].wait()
        outs[k].start()
    for c in outs:
        c.wait()


@functools.cache
def _make_copy_kernel():
    return pl.pallas_call(
        _copy_body,
        in_specs=[pl.BlockSpec(memory_space=pl.ANY)],
        out_specs=pl.BlockSpec(memory_space=pl.ANY),
        out_shape=jax.ShapeDtypeStruct((_COLS, _ROWS), jnp.float32),
        scratch_shapes=[pltpu.VMEM((_COLS, _ROWS), jnp.float32)]
        + [pltpu.SemaphoreType.DMA] * (2 * _NCH),
    )


def kernel(candidate_titles):
    xt = pltpu.with_memory_space_constraint(
        candidate_titles.T, pltpu.MemorySpace.HBM
    )
    return _make_copy_kernel()(xt).T


# speedup vs baseline: 1.0280x; 1.0003x over previous
import functools

import jax
import jax.numpy as jnp
from jax.experimental import pallas as pl
from jax.experimental.pallas import tpu as pltpu

_ROWS, _COLS = 16384, 50
_RCHUNKS = ((48, 2), (0, 8), (8, 8), (16, 8), (24, 8), (32, 8), (40, 8))
_NCH = len(_RCHUNKS)


def _copy_body(x_ref, o_ref, buf, *sems):
    in_sems, out_sems = sems[:_NCH], sems[_NCH:]
    ins = [
        pltpu.make_async_copy(
            x_ref.at[pl.ds(r0, nr)],
            buf.at[pl.ds(r0, nr)],
            in_sems[k],
        )
        for k, (r0, nr) in enumerate(_RCHUNKS)
    ]
    outs = [
        pltpu.make_async_copy(
            buf.at[pl.ds(r0, nr)],
            o_ref.at[pl.ds(r0, nr)],
            out_sems[k],
        )
        for k, (r0, nr) in enumerate(_RCHUNKS)
    ]
    for c in ins:
        c.start()
    for k in range(_NCH):
        ins[k].wait()
        outs[k].start()
    for c in outs:
        c.wait()


@functools.cache
def _make_copy_kernel():
    return pl.pallas_call(
        _copy_body,
        in_specs=[pl.BlockSpec(memory_space=pl.ANY)],
        out_specs=pl.BlockSpec(memory_space=pl.ANY),
        out_shape=jax.ShapeDtypeStruct((_COLS, _ROWS), jnp.float32),
        scratch_shapes=[pltpu.VMEM((_COLS, _ROWS), jnp.float32)]
        + [pltpu.SemaphoreType.DMA] * (2 * _NCH),
    )


def kernel(candidate_titles):
    xt = pltpu.with_memory_space_constraint(
        candidate_titles.T, pltpu.MemorySpace.HBM
    )
    return _make_copy_kernel()(xt).T


# speedup vs baseline: 1.0377x; 1.0095x over previous
import functools

import jax
import jax.numpy as jnp
from jax.experimental import pallas as pl
from jax.experimental.pallas import tpu as pltpu

_ROWS, _COLS = 16384, 50
_RCHUNKS = ((0, 8), (8, 8), (16, 8), (24, 8), (32, 8), (40, 8), (48, 2))
_NCH = len(_RCHUNKS)


def _copy_body(x_ref, o_ref, buf, *sems):
    in_sems, out_sems = sems[:_NCH], sems[_NCH:]
    ins = [
        pltpu.make_async_copy(
            x_ref.at[pl.ds(r0, nr)],
            buf.at[pl.ds(r0, nr)],
            in_sems[k],
        )
        for k, (r0, nr) in enumerate(_RCHUNKS)
    ]
    outs = [
        pltpu.make_async_copy(
            buf.at[pl.ds(r0, nr)],
            o_ref.at[pl.ds(r0, nr)],
            out_sems[k],
        )
        for k, (r0, nr) in enumerate(_RCHUNKS)
    ]
    for c in ins:
        c.start()
    for k in range(_NCH):
        ins[k].wait()
        outs[k].start()
    for c in outs:
        c.wait()


@functools.cache
def _make_copy_kernel():
    return pl.pallas_call(
        _copy_body,
        in_specs=[pl.BlockSpec(memory_space=pl.ANY)],
        out_specs=pl.BlockSpec(memory_space=pl.ANY),
        out_shape=jax.ShapeDtypeStruct((_COLS, _ROWS), jnp.float32),
        scratch_shapes=[pltpu.VMEM((_COLS, _ROWS), jnp.float32)]
        + [pltpu.SemaphoreType.DMA] * (2 * _NCH),
    )


def kernel(candidate_titles):
    xt = pltpu.with_memory_space_constraint(
        candidate_titles.T, pltpu.MemorySpace.HBM
    )
    return _make_copy_kernel()(xt).T
